# baseline (speedup 1.0000x reference)
import jax
import jax.numpy as jnp
from jax import lax
from jax.experimental import pallas as pl

TOP_K = 64
TQ = 256


def _attn_block(q_ref, k_ref, v_ref, o_ref):
    q = q_ref[0]
    k = k_ref[0]
    v = v_ref[0]
    nq = q.shape[0]
    lk = k.shape[0]

    s = lax.dot_general(q, k, (((1,), (1,)), ((), ())),
                        preferred_element_type=jnp.float32)

    i = lax.bitcast_convert_type(s, jnp.int32)
    key = jnp.where(i >= 0, i, i ^ jnp.int32(0x7FFFFFFF))

    ones_f = jnp.ones((lk, 8), jnp.float32)

    def count(mask):
        self_ = jnp.where(mask, 1.0, 0.0)
        return lax.dot_general(self_, ones_f, (((1,), (0,)), ((), ())),
                               preferred_element_type=jnp.float32)[:, :1]

    kk = jnp.float32(TOP_K)
    zero32 = jnp.zeros((nq, 1), jnp.int32)

    t = jnp.where(count(key >= zero32) >= kk,
                  jnp.int32(0), jnp.int32(-2147483648))
    for bit in range(30, -1, -1):
        cand = t + jnp.int32(1 << bit)
        t = jnp.where(count(key >= cand) >= kk, cand, t)

    sel = key >= t
    m = jnp.max(s, axis=1, keepdims=True)
    e = jnp.where(sel, jnp.exp(s - m), 0.0)
    z = jnp.sum(e, axis=1, keepdims=True)
    p = e * (1.0 / z)

    o_ref[0] = lax.dot_general(p, v, (((1,), (0,)), ((), ())),
                               preferred_element_type=jnp.float32)


@jax.jit
def kernel(Q, K, V):
    B, Lq, d = Q.shape
    Lk = K.shape[1]
    grid = (B, Lq // TQ)
    return pl.pallas_call(
        _attn_block,
        grid=grid,
        in_specs=[
            pl.BlockSpec((1, TQ, d), lambda b, i: (b, i, 0)),
            pl.BlockSpec((1, Lk, d), lambda b, i: (b, 0, 0)),
            pl.BlockSpec((1, Lk, d), lambda b, i: (b, 0, 0)),
        ],
        out_specs=pl.BlockSpec((1, TQ, d), lambda b, i: (b, i, 0)),
        out_shape=jax.ShapeDtypeStruct((B, Lq, d), jnp.float32),
    )(Q, K, V)


# speedup vs baseline: 1.4311x; 1.4311x over previous
import jax
import jax.numpy as jnp
from jax import lax
from jax.experimental import pallas as pl

TOP_K = 64
TQ = 256


def _attn_block(q_ref, k_ref, v_ref, o_ref):
    q = q_ref[0]
    k = k_ref[0]
    v = v_ref[0]
    nq = q.shape[0]
    lk = k.shape[0]

    s = lax.dot_general(q, k, (((1,), (1,)), ((), ())),
                        preferred_element_type=jnp.float32)

    i = lax.bitcast_convert_type(s, jnp.int32)
    key = jnp.where(i >= 0, i, i ^ jnp.int32(0x7FFFFFFF))
    khi = (key >> 16).astype(jnp.int16)
    klo = ((key & 0xFFFF) ^ 0x8000).astype(jnp.int16)

    ones_b = jnp.ones((lk, 8), jnp.bfloat16)
    one_b = jnp.bfloat16(1)
    zero_b = jnp.bfloat16(0)

    def count(mask16):
        selb = jnp.where(mask16, one_b, zero_b)
        return lax.dot_general(selb, ones_b, (((1,), (0,)), ((), ())),
                               preferred_element_type=jnp.float32)[:, :1]

    def cast16(c32):
        return c32.astype(jnp.int16)

    kk = jnp.float32(TOP_K)
    neg_inf16 = jnp.int32(-32768)

    t_hi32 = jnp.where(count(khi >= cast16(jnp.zeros((nq, 1), jnp.int32)))
                       >= kk, jnp.int32(0), neg_inf16)
    for bit in range(14, -1, -1):
        cand32 = t_hi32 + jnp.int32(1 << bit)
        t_hi32 = jnp.where(count(khi >= cast16(cand32)) >= kk,
                           cand32, t_hi32)
    t_hi = cast16(t_hi32)

    gt = khi > t_hi
    eq = khi == t_hi
    r = kk - count(gt)

    klo_m = jnp.where(eq, klo, jnp.int16(-32768))
    t_lo32 = jnp.where(count(klo_m >= cast16(jnp.zeros((nq, 1), jnp.int32)))
                       >= r, jnp.int32(0), neg_inf16)
    for bit in range(14, -1, -1):
        cand32 = t_lo32 + jnp.int32(1 << bit)
        t_lo32 = jnp.where(count(klo_m >= cast16(cand32)) >= r,
                           cand32, t_lo32)

    t32 = (t_hi32 << 16) + (t_lo32 + jnp.int32(32768))
    sel = key >= t32
    m = jnp.max(s, axis=1, keepdims=True)
    e = jnp.where(sel, jnp.exp(s - m), 0.0)
    z = jnp.sum(e, axis=1, keepdims=True)
    p = e * (1.0 / z)

    o_ref[0] = lax.dot_general(p, v, (((1,), (0,)), ((), ())),
                               preferred_element_type=jnp.float32)


@jax.jit
def kernel(Q, K, V):
    B, Lq, d = Q.shape
    Lk = K.shape[1]
    grid = (B, Lq // TQ)
    return pl.pallas_call(
        _attn_block,
        grid=grid,
        in_specs=[
            pl.BlockSpec((1, TQ, d), lambda b, i: (b, i, 0)),
            pl.BlockSpec((1, Lk, d), lambda b, i: (b, 0, 0)),
            pl.BlockSpec((1, Lk, d), lambda b, i: (b, 0, 0)),
        ],
        out_specs=pl.BlockSpec((1, TQ, d), lambda b, i: (b, i, 0)),
        out_shape=jax.ShapeDtypeStruct((B, Lq, d), jnp.float32),
    )(Q, K, V)


# speedup vs baseline: 2.0423x; 1.4271x over previous
import jax
import jax.numpy as jnp
from jax import lax
from jax.experimental import pallas as pl

TOP_K = 64
TQ = 256
N_CHAINS = 4


def _attn_block(q_ref, k_ref, v_ref, o_ref):
    q = q_ref[0]
    k = k_ref[0]
    v = v_ref[0]
    lk = k.shape[0]

    s_full = lax.dot_general(q, k, (((1,), (1,)), ((), ())),
                             preferred_element_type=jnp.float32)

    ones_b = jnp.ones((lk, 8), jnp.bfloat16)
    one_b = jnp.bfloat16(1)
    zero_b = jnp.bfloat16(0)
    kk = jnp.float32(TOP_K)
    neg_inf16 = jnp.int32(-32768)
    rows = TQ // N_CHAINS
    C = N_CHAINS

    def count(mask16):
        selb = jnp.where(mask16, one_b, zero_b)
        return lax.dot_general(selb, ones_b, (((1,), (0,)), ((), ())),
                               preferred_element_type=jnp.float32)[:, :1]

    def cast16(c32):
        return c32.astype(jnp.int16)

    s = [s_full[c * rows:(c + 1) * rows] for c in range(C)]
    key, khi, klo = [], [], []
    for c in range(C):
        i = lax.bitcast_convert_type(s[c], jnp.int32)
        kc = jnp.where(i >= 0, i, i ^ jnp.int32(0x7FFFFFFF))
        key.append(kc)
        khi.append((kc >> 16).astype(jnp.int16))
        klo.append(((kc & 0xFFFF) ^ 0x8000).astype(jnp.int16))

    zeros16 = [cast16(jnp.zeros((rows, 1), jnp.int32)) for _ in range(C)]

    t_hi32 = [jnp.where(count(khi[c] >= zeros16[c]) >= kk,
                        jnp.int32(0), neg_inf16) for c in range(C)]
    for bit in range(14, -1, -1):
        cand = [t_hi32[c] + jnp.int32(1 << bit) for c in range(C)]
        cnt = [count(khi[c] >= cast16(cand[c])) for c in range(C)]
        t_hi32 = [jnp.where(cnt[c] >= kk, cand[c], t_hi32[c])
                  for c in range(C)]
    t_hi = [cast16(t_hi32[c]) for c in range(C)]

    gt = [khi[c] > t_hi[c] for c in range(C)]
    eq = [khi[c] == t_hi[c] for c in range(C)]
    r = [kk - count(gt[c]) for c in range(C)]

    klo_m = [jnp.where(eq[c], klo[c], jnp.int16(-32768)) for c in range(C)]
    t_lo32 = [jnp.where(count(klo_m[c] >= zeros16[c]) >= r[c],
                        jnp.int32(0), neg_inf16) for c in range(C)]
    for bit in range(14, -1, -1):
        cand = [t_lo32[c] + jnp.int32(1 << bit) for c in range(C)]
        cnt = [count(klo_m[c] >= cast16(cand[c])) for c in range(C)]
        t_lo32 = [jnp.where(cnt[c] >= r[c], cand[c], t_lo32[c])
                  for c in range(C)]

    outs = []
    for c in range(C):
        t32 = (t_hi32[c] << 16) + (t_lo32[c] + jnp.int32(32768))
        sel = key[c] >= t32
        m = jnp.max(s[c], axis=1, keepdims=True)
        e = jnp.where(sel, jnp.exp(s[c] - m), 0.0)
        z = jnp.sum(e, axis=1, keepdims=True)
        p = e * (1.0 / z)
        outs.append(lax.dot_general(p, v, (((1,), (0,)), ((), ())),
                                    preferred_element_type=jnp.float32))
    o_ref[0] = jnp.concatenate(outs, axis=0)


@jax.jit
def kernel(Q, K, V):
    B, Lq, d = Q.shape
    Lk = K.shape[1]
    grid = (B, Lq // TQ)
    return pl.pallas_call(
        _attn_block,
        grid=grid,
        in_specs=[
            pl.BlockSpec((1, TQ, d), lambda b, i: (b, i, 0)),
            pl.BlockSpec((1, Lk, d), lambda b, i: (b, 0, 0)),
            pl.BlockSpec((1, Lk, d), lambda b, i: (b, 0, 0)),
        ],
        out_specs=pl.BlockSpec((1, TQ, d), lambda b, i: (b, i, 0)),
        out_shape=jax.ShapeDtypeStruct((B, Lq, d), jnp.float32),
    )(Q, K, V)


# speedup vs baseline: 2.1312x; 1.0435x over previous
import jax
import jax.numpy as jnp
from jax import lax
from jax.experimental import pallas as pl

TOP_K = 64
TQ = 512
N_CHAINS = 8


def _attn_block(q_ref, k_ref, v_ref, o_ref):
    q = q_ref[0]
    k = k_ref[0]
    v = v_ref[0]
    lk = k.shape[0]

    s_full = lax.dot_general(q, k, (((1,), (1,)), ((), ())),
                             preferred_element_type=jnp.float32)

    ones_b = jnp.ones((lk, 8), jnp.bfloat16)
    one_b = jnp.bfloat16(1)
    zero_b = jnp.bfloat16(0)
    kk = jnp.float32(TOP_K)
    neg_inf16 = jnp.int32(-32768)
    rows = TQ // N_CHAINS
    C = N_CHAINS

    def count(mask16):
        selb = jnp.where(mask16, one_b, zero_b)
        return lax.dot_general(selb, ones_b, (((1,), (0,)), ((), ())),
                               preferred_element_type=jnp.float32)[:, :1]

    def cast16(c32):
        return c32.astype(jnp.int16)

    s = [s_full[c * rows:(c + 1) * rows] for c in range(C)]
    key, khi, klo = [], [], []
    for c in range(C):
        i = lax.bitcast_convert_type(s[c], jnp.int32)
        kc = jnp.where(i >= 0, i, i ^ jnp.int32(0x7FFFFFFF))
        key.append(kc)
        khi.append((kc >> 16).astype(jnp.int16))
        klo.append(((kc & 0xFFFF) ^ 0x8000).astype(jnp.int16))

    zeros16 = [cast16(jnp.zeros((rows, 1), jnp.int32)) for _ in range(C)]

    t_hi32 = [jnp.where(count(khi[c] >= zeros16[c]) >= kk,
                        jnp.int32(0), neg_inf16) for c in range(C)]
    for bit in range(14, -1, -1):
        cand = [t_hi32[c] + jnp.int32(1 << bit) for c in range(C)]
        cnt = [count(khi[c] >= cast16(cand[c])) for c in range(C)]
        t_hi32 = [jnp.where(cnt[c] >= kk, cand[c], t_hi32[c])
                  for c in range(C)]
    t_hi = [cast16(t_hi32[c]) for c in range(C)]

    klo_m = [jnp.where(khi[c] > t_hi[c], jnp.int16(32767),
                       jnp.where(khi[c] == t_hi[c], klo[c],
                                 jnp.int16(-32768))) for c in range(C)]
    t_lo32 = [jnp.where(count(klo_m[c] >= zeros16[c]) >= kk,
                        jnp.int32(0), neg_inf16) for c in range(C)]
    for bit in range(14, -1, -1):
        cand = [t_lo32[c] + jnp.int32(1 << bit) for c in range(C)]
        cnt = [count(klo_m[c] >= cast16(cand[c])) for c in range(C)]
        t_lo32 = [jnp.where(cnt[c] >= kk, cand[c], t_lo32[c])
                  for c in range(C)]

    outs = []
    for c in range(C):
        t32 = (t_hi32[c] << 16) + (t_lo32[c] + jnp.int32(32768))
        sel = key[c] >= t32
        m = jnp.max(s[c], axis=1, keepdims=True)
        e = jnp.where(sel, jnp.exp(s[c] - m), 0.0)
        z = jnp.sum(e, axis=1, keepdims=True)
        p = e * (1.0 / z)
        outs.append(lax.dot_general(p, v, (((1,), (0,)), ((), ())),
                                    preferred_element_type=jnp.float32))
    o_ref[0] = jnp.concatenate(outs, axis=0)


@jax.jit
def kernel(Q, K, V):
    B, Lq, d = Q.shape
    Lk = K.shape[1]
    grid = (B, Lq // TQ)
    return pl.pallas_call(
        _attn_block,
        grid=grid,
        in_specs=[
            pl.BlockSpec((1, TQ, d), lambda b, i: (b, i, 0)),
            pl.BlockSpec((1, Lk, d), lambda b, i: (b, 0, 0)),
            pl.BlockSpec((1, Lk, d), lambda b, i: (b, 0, 0)),
        ],
        out_specs=pl.BlockSpec((1, TQ, d), lambda b, i: (b, i, 0)),
        out_shape=jax.ShapeDtypeStruct((B, Lq, d), jnp.float32),
    )(Q, K, V)


# speedup vs baseline: 2.4550x; 1.1519x over previous
import jax
import jax.numpy as jnp
from jax import lax
from jax.experimental import pallas as pl

TOP_K = 64
TQ = 512
N_CHAINS = 8


def _attn_block(q_ref, k_ref, v_ref, o_ref):
    q = q_ref[0]
    k = k_ref[0]
    v = v_ref[0]
    lk = k.shape[0]

    s_full = lax.dot_general(q, k, (((1,), (1,)), ((), ())),
                             preferred_element_type=jnp.float32)

    ones_b = jnp.ones((lk, 8), jnp.bfloat16)
    one_b = jnp.bfloat16(1)
    zero_b = jnp.bfloat16(0)
    kk = jnp.float32(TOP_K)
    neg_inf16 = jnp.int32(-32768)
    rows = TQ // N_CHAINS
    C = N_CHAINS

    def count_mxu(mask16):
        selb = jnp.where(mask16, one_b, zero_b)
        return lax.dot_general(selb, ones_b, (((1,), (0,)), ((), ())),
                               preferred_element_type=jnp.float32)[:, :1]

    def count_vpu(mask16):
        selb = jnp.where(mask16, one_b, zero_b).astype(jnp.float32)
        return jnp.sum(selb, axis=1, keepdims=True)

    counts = [count_mxu if c % 2 == 0 else count_vpu for c in range(N_CHAINS)]

    def count_c(c, mask16):
        return counts[c](mask16)

    def cast16(c32):
        return c32.astype(jnp.int16)

    s = [s_full[c * rows:(c + 1) * rows] for c in range(C)]
    key, khi, klo = [], [], []
    for c in range(C):
        i = lax.bitcast_convert_type(s[c], jnp.int32)
        kc = jnp.where(i >= 0, i, i ^ jnp.int32(0x7FFFFFFF))
        key.append(kc)
        khi.append((kc >> 16).astype(jnp.int16))
        klo.append(((kc & 0xFFFF) ^ 0x8000).astype(jnp.int16))

    zeros16 = [cast16(jnp.zeros((rows, 1), jnp.int32)) for _ in range(C)]

    t_hi32 = [jnp.where(count_c(c, khi[c] >= zeros16[c]) >= kk,
                        jnp.int32(0), neg_inf16) for c in range(C)]
    for bit in range(14, -1, -1):
        cand = [t_hi32[c] + jnp.int32(1 << bit) for c in range(C)]
        cnt = [count_c(c, khi[c] >= cast16(cand[c])) for c in range(C)]
        t_hi32 = [jnp.where(cnt[c] >= kk, cand[c], t_hi32[c])
                  for c in range(C)]
    t_hi = [cast16(t_hi32[c]) for c in range(C)]

    klo_m = [jnp.where(khi[c] > t_hi[c], jnp.int16(32767),
                       jnp.where(khi[c] == t_hi[c], klo[c],
                                 jnp.int16(-32768))) for c in range(C)]
    t_lo32 = [jnp.where(count_c(c, klo_m[c] >= zeros16[c]) >= kk,
                        jnp.int32(0), neg_inf16) for c in range(C)]
    for bit in range(14, -1, -1):
        cand = [t_lo32[c] + jnp.int32(1 << bit) for c in range(C)]
        cnt = [count_c(c, klo_m[c] >= cast16(cand[c])) for c in range(C)]
        t_lo32 = [jnp.where(cnt[c] >= kk, cand[c], t_lo32[c])
                  for c in range(C)]

    outs = []
    for c in range(C):
        t32 = (t_hi32[c] << 16) + (t_lo32[c] + jnp.int32(32768))
        sel = key[c] >= t32
        m = jnp.max(s[c], axis=1, keepdims=True)
        e = jnp.where(sel, jnp.exp(s[c] - m), 0.0)
        z = jnp.sum(e, axis=1, keepdims=True)
        p = e * (1.0 / z)
        outs.append(lax.dot_general(p, v, (((1,), (0,)), ((), ())),
                                    preferred_element_type=jnp.float32))
    o_ref[0] = jnp.concatenate(outs, axis=0)


@jax.jit
def kernel(Q, K, V):
    B, Lq, d = Q.shape
    Lk = K.shape[1]
    grid = (B, Lq // TQ)
    return pl.pallas_call(
        _attn_block,
        grid=grid,
        in_specs=[
            pl.BlockSpec((1, TQ, d), lambda b, i: (b, i, 0)),
            pl.BlockSpec((1, Lk, d), lambda b, i: (b, 0, 0)),
            pl.BlockSpec((1, Lk, d), lambda b, i: (b, 0, 0)),
        ],
        out_specs=pl.BlockSpec((1, TQ, d), lambda b, i: (b, i, 0)),
        out_shape=jax.ShapeDtypeStruct((B, Lq, d), jnp.float32),
    )(Q, K, V)


# speedup vs baseline: 3.4671x; 1.4123x over previous
import jax
import jax.numpy as jnp
from jax import lax
from jax.experimental import pallas as pl

TOP_K = 64
TQ = 512
N_CHAINS = 8


def _attn_block(q_ref, k_ref, v_ref, o_ref):
    q = q_ref[0]
    k = k_ref[0]
    v = v_ref[0]
    lk = k.shape[0]

    s_full = lax.dot_general(q, k, (((1,), (1,)), ((), ())),
                             preferred_element_type=jnp.float32)

    ones_b = jnp.ones((128, 8), jnp.bfloat16)
    one_b = jnp.bfloat16(1)
    zero_b = jnp.bfloat16(0)
    kk = jnp.float32(TOP_K)
    neg_inf16 = jnp.int32(-32768)
    rows = TQ // N_CHAINS
    C = N_CHAINS

    def lane_tree(mask16):
        x = jnp.where(mask16, one_b, zero_b)
        w = lk
        while w > 128:
            h = w // 2
            x = x[:, :h] + x[:, h:w]
            w = h
        return x

    def count_mxu(mask16):
        return lax.dot_general(lane_tree(mask16), ones_b,
                               (((1,), (0,)), ((), ())),
                               preferred_element_type=jnp.float32)[:, :1]

    def count_vpu(mask16):
        return jnp.sum(lane_tree(mask16).astype(jnp.float32),
                       axis=1, keepdims=True)

    counts = [count_mxu if c % 2 == 0 else count_vpu for c in range(N_CHAINS)]

    def count_c(c, mask16):
        return counts[c](mask16)

    def cast16(c32):
        return c32.astype(jnp.int16)

    s = [s_full[c * rows:(c + 1) * rows] for c in range(C)]
    key, khi, klo = [], [], []
    for c in range(C):
        i = lax.bitcast_convert_type(s[c], jnp.int32)
        kc = jnp.where(i >= 0, i, i ^ jnp.int32(0x7FFFFFFF))
        key.append(kc)
        khi.append((kc >> 16).astype(jnp.int16))
        klo.append(((kc & 0xFFFF) ^ 0x8000).astype(jnp.int16))

    zeros16 = [cast16(jnp.zeros((rows, 1), jnp.int32)) for _ in range(C)]

    t_hi32 = [jnp.where(count_c(c, khi[c] >= zeros16[c]) >= kk,
                        jnp.int32(0), neg_inf16) for c in range(C)]
    for bit in range(14, -1, -1):
        cand = [t_hi32[c] + jnp.int32(1 << bit) for c in range(C)]
        cnt = [count_c(c, khi[c] >= cast16(cand[c])) for c in range(C)]
        t_hi32 = [jnp.where(cnt[c] >= kk, cand[c], t_hi32[c])
                  for c in range(C)]
    t_hi = [cast16(t_hi32[c]) for c in range(C)]

    klo_m = [jnp.where(khi[c] > t_hi[c], jnp.int16(32767),
                       jnp.where(khi[c] == t_hi[c], klo[c],
                                 jnp.int16(-32768))) for c in range(C)]
    t_lo32 = [jnp.where(count_c(c, klo_m[c] >= zeros16[c]) >= kk,
                        jnp.int32(0), neg_inf16) for c in range(C)]
    for bit in range(14, -1, -1):
        cand = [t_lo32[c] + jnp.int32(1 << bit) for c in range(C)]
        cnt = [count_c(c, klo_m[c] >= cast16(cand[c])) for c in range(C)]
        t_lo32 = [jnp.where(cnt[c] >= kk, cand[c], t_lo32[c])
                  for c in range(C)]

    outs = []
    for c in range(C):
        t32 = (t_hi32[c] << 16) + (t_lo32[c] + jnp.int32(32768))
        sel = key[c] >= t32
        m = jnp.max(s[c], axis=1, keepdims=True)
        e = jnp.where(sel, jnp.exp(s[c] - m), 0.0)
        z = jnp.sum(e, axis=1, keepdims=True)
        p = e * (1.0 / z)
        outs.append(lax.dot_general(p, v, (((1,), (0,)), ((), ())),
                                    preferred_element_type=jnp.float32))
    o_ref[0] = jnp.concatenate(outs, axis=0)


@jax.jit
def kernel(Q, K, V):
    B, Lq, d = Q.shape
    Lk = K.shape[1]
    grid = (B, Lq // TQ)
    return pl.pallas_call(
        _attn_block,
        grid=grid,
        in_specs=[
            pl.BlockSpec((1, TQ, d), lambda b, i: (b, i, 0)),
            pl.BlockSpec((1, Lk, d), lambda b, i: (b, 0, 0)),
            pl.BlockSpec((1, Lk, d), lambda b, i: (b, 0, 0)),
        ],
        out_specs=pl.BlockSpec((1, TQ, d), lambda b, i: (b, i, 0)),
        out_shape=jax.ShapeDtypeStruct((B, Lq, d), jnp.float32),
    )(Q, K, V)


# speedup vs baseline: 3.8555x; 1.1120x over previous
import jax
import jax.numpy as jnp
from jax import lax
from jax.experimental import pallas as pl

TOP_K = 64
TQ = 1024
N_CHAINS = 16


def _attn_block(q_ref, k_ref, v_ref, o_ref):
    q = q_ref[0]
    k = k_ref[0]
    v = v_ref[0]
    lk = k.shape[0]

    s_full = lax.dot_general(q, k, (((1,), (1,)), ((), ())),
                             preferred_element_type=jnp.float32)

    ones_b = jnp.ones((128, 8), jnp.bfloat16)
    one_b = jnp.bfloat16(1)
    zero_b = jnp.bfloat16(0)
    kk = jnp.float32(TOP_K)
    neg_inf16 = jnp.int32(-32768)
    rows = TQ // N_CHAINS
    C = N_CHAINS

    def lane_tree(mask16):
        x = jnp.where(mask16, one_b, zero_b)
        w = lk
        while w > 128:
            h = w // 2
            x = x[:, :h] + x[:, h:w]
            w = h
        return x

    def count_mxu(mask16):
        return lax.dot_general(lane_tree(mask16), ones_b,
                               (((1,), (0,)), ((), ())),
                               preferred_element_type=jnp.float32)[:, :1]

    def count_vpu(mask16):
        return jnp.sum(lane_tree(mask16).astype(jnp.float32),
                       axis=1, keepdims=True)

    counts = [count_mxu if c % 2 == 0 else count_vpu for c in range(N_CHAINS)]

    def count_c(c, mask16):
        return counts[c](mask16)

    def cast16(c32):
        return c32.astype(jnp.int16)

    s = [s_full[c * rows:(c + 1) * rows] for c in range(C)]
    khi, klo = [], []
    for c in range(C):
        i = lax.bitcast_convert_type(s[c], jnp.int32)
        kc = jnp.where(i >= 0, i, i ^ jnp.int32(0x7FFFFFFF))
        khi.append((kc >> 16).astype(jnp.int16))
        klo.append(((kc & 0xFFFF) ^ 0x8000).astype(jnp.int16))

    zeros16 = [cast16(jnp.zeros((rows, 1), jnp.int32)) for _ in range(C)]

    t_hi32 = [jnp.where(count_c(c, khi[c] >= zeros16[c]) >= kk,
                        jnp.int32(0), neg_inf16) for c in range(C)]
    for bit in range(14, -1, -1):
        cand = [t_hi32[c] + jnp.int32(1 << bit) for c in range(C)]
        cnt = [count_c(c, khi[c] >= cast16(cand[c])) for c in range(C)]
        t_hi32 = [jnp.where(cnt[c] >= kk, cand[c], t_hi32[c])
                  for c in range(C)]
    t_hi = [cast16(t_hi32[c]) for c in range(C)]

    klo_m = [jnp.where(khi[c] > t_hi[c], jnp.int16(32767),
                       jnp.where(khi[c] == t_hi[c], klo[c],
                                 jnp.int16(-32768))) for c in range(C)]
    t_lo32 = [jnp.where(count_c(c, klo_m[c] >= zeros16[c]) >= kk,
                        jnp.int32(0), neg_inf16) for c in range(C)]
    for bit in range(14, -1, -1):
        cand = [t_lo32[c] + jnp.int32(1 << bit) for c in range(C)]
        cnt = [count_c(c, klo_m[c] >= cast16(cand[c])) for c in range(C)]
        t_lo32 = [jnp.where(cnt[c] >= kk, cand[c], t_lo32[c])
                  for c in range(C)]

    outs = []
    for c in range(C):
        t32 = (t_hi32[c] << 16) + (t_lo32[c] + jnp.int32(32768))
        tbits = jnp.where(t32 >= 0, t32, t32 ^ jnp.int32(0x7FFFFFFF))
        tf = lax.bitcast_convert_type(tbits, jnp.float32)
        sel = s[c] >= tf
        m = jnp.max(s[c], axis=1, keepdims=True)
        e = jnp.where(sel, jnp.exp(s[c] - m), 0.0)
        z = jnp.sum(e, axis=1, keepdims=True)
        p = e * (1.0 / z)
        outs.append(lax.dot_general(p, v, (((1,), (0,)), ((), ())),
                                    preferred_element_type=jnp.float32))
    o_ref[0] = jnp.concatenate(outs, axis=0)


@jax.jit
def kernel(Q, K, V):
    B, Lq, d = Q.shape
    Lk = K.shape[1]
    grid = (B, Lq // TQ)
    return pl.pallas_call(
        _attn_block,
        grid=grid,
        in_specs=[
            pl.BlockSpec((1, TQ, d), lambda b, i: (b, i, 0)),
            pl.BlockSpec((1, Lk, d), lambda b, i: (b, 0, 0)),
            pl.BlockSpec((1, Lk, d), lambda b, i: (b, 0, 0)),
        ],
        out_specs=pl.BlockSpec((1, TQ, d), lambda b, i: (b, i, 0)),
        out_shape=jax.ShapeDtypeStruct((B, Lq, d), jnp.float32),
    )(Q, K, V)
